# Initial kernel scaffold; baseline (speedup 1.0000x reference)
import jax
import jax.numpy as jnp
from jax.experimental import pallas as pl


def kernel(x, H, weight, bias, Wn_w, Wn_b, comp_w, comp_b, he_bias):
    raise NotImplementedError("write your pallas kernel here")


# speedup vs baseline: 2.0050x; 2.0050x over previous
import functools
import math

import jax
import jax.numpy as jnp
from jax.experimental import pallas as pl
from jax.experimental.pallas import tpu as pltpu


def _bf16_dot(a, b, dims):
    return jax.lax.dot_general(
        a.astype(jnp.bfloat16), b.astype(jnp.bfloat16), dims,
        preferred_element_type=jnp.float32)


def _fused_body(x_ref, h_ref, w_ref, b_ref, wn_ref, wnb_ref, cw_ref, cb_ref,
                o_ref, *, scale):
    xb = x_ref[0]
    n = xb.shape[0]
    hf = h_ref[...].astype(jnp.float32)

    de = jnp.maximum(jnp.sum(hf, axis=0), 1.0)
    dv = jnp.maximum(jnp.sum(hf, axis=1), 1.0)

    edge_feat = _bf16_dot(hf, xb, (((0,), (0,)), ((), ())))
    edge_feat = edge_feat / de[:, None]

    logit = jnp.sum(edge_feat * cw_ref[...], axis=1, keepdims=True)
    gamma = jax.nn.sigmoid(logit + cb_ref[0, 0])

    x_hyper = _bf16_dot(hf, gamma * edge_feat, (((1,), (0,)), ((), ())))
    x_hyper = x_hyper / dv[:, None]

    xp = _bf16_dot(xb, wn_ref[...], (((1,), (0,)), ((), ()))) + wnb_ref[...]
    scores = _bf16_dot(xp, xp, (((1,), (1,)), ((), ()))) * scale
    sqn = jnp.sqrt(jnp.sum(xp * xp, axis=1, keepdims=True))
    bound = sqn * (jnp.max(sqn) * scale)
    e = jnp.exp(scores - bound)
    num = _bf16_dot(e, xb, (((1,), (0,)), ((), ())))
    ones = jnp.ones((n, 1), dtype=jnp.bfloat16)
    den = _bf16_dot(e, ones, (((1,), (0,)), ((), ())))
    x_node = num / den

    out = _bf16_dot(x_hyper + x_node, w_ref[...],
                    (((1,), (0,)), ((), ()))) + b_ref[...]
    o_ref[0] = out


def kernel(x, H, weight, bias, Wn_w, Wn_b, comp_w, comp_b, he_bias):
    B, N, F = x.shape
    O = weight.shape[1]
    E = H.shape[1]
    scale = 1.0 / math.sqrt(F)

    bias2 = bias.reshape(1, O)
    wnb2 = Wn_b.reshape(1, F)
    cw2 = comp_w.reshape(1, F)
    cb2 = (comp_b + he_bias).reshape(1, 1)

    grid = (B,)
    out = pl.pallas_call(
        functools.partial(_fused_body, scale=scale),
        grid=grid,
        in_specs=[
            pl.BlockSpec((1, N, F), lambda b: (b, 0, 0)),
            pl.BlockSpec((N, E), lambda b: (0, 0)),
            pl.BlockSpec((F, O), lambda b: (0, 0)),
            pl.BlockSpec((1, O), lambda b: (0, 0)),
            pl.BlockSpec((F, F), lambda b: (0, 0)),
            pl.BlockSpec((1, F), lambda b: (0, 0)),
            pl.BlockSpec((1, F), lambda b: (0, 0)),
            pl.BlockSpec((1, 1), lambda b: (0, 0)),
        ],
        out_specs=pl.BlockSpec((1, N, O), lambda b: (b, 0, 0)),
        out_shape=jax.ShapeDtypeStruct((B, N, O), jnp.float32),
        compiler_params=pltpu.CompilerParams(
            dimension_semantics=("arbitrary",),
            vmem_limit_bytes=128 * 1024 * 1024,
        ),
    )(x, H, weight, bias2, Wn_w, wnb2, cw2, cb2)
    return out


# speedup vs baseline: 2.3582x; 1.1762x over previous
import functools
import math

import jax
import jax.numpy as jnp
from jax.experimental import pallas as pl
from jax.experimental.pallas import tpu as pltpu


def _bf16_dot(a, b, dims):
    return jax.lax.dot_general(
        a.astype(jnp.bfloat16), b.astype(jnp.bfloat16), dims,
        preferred_element_type=jnp.float32)


def _fused_body(x_ref, h_ref, w_ref, b_ref, wn_ref, wnb_ref, cw_ref, cb_ref,
                o_ref, *, scale):
    xb = x_ref[0]
    n = xb.shape[0]
    hf = h_ref[...].astype(jnp.float32)

    de = jnp.maximum(jnp.sum(hf, axis=0), 1.0)
    dv = jnp.maximum(jnp.sum(hf, axis=1), 1.0)

    edge_feat = _bf16_dot(hf, xb, (((0,), (0,)), ((), ())))
    edge_feat = edge_feat / de[:, None]

    logit = jnp.sum(edge_feat * cw_ref[...], axis=1, keepdims=True)
    gamma = jax.nn.sigmoid(logit + cb_ref[0, 0])

    x_hyper = _bf16_dot(hf, gamma * edge_feat, (((1,), (0,)), ((), ())))
    x_hyper = x_hyper / dv[:, None]

    xp = _bf16_dot(xb, wn_ref[...], (((1,), (0,)), ((), ()))) + wnb_ref[...]
    log2e_scale = scale * 1.4426950408889634
    sqn2 = jnp.sum(xp * xp, axis=1, keepdims=True)
    bound2 = jnp.sqrt(sqn2 * jnp.max(sqn2)) * log2e_scale
    xq = (xp * log2e_scale).astype(jnp.bfloat16)
    s2 = jax.lax.dot_general(
        xq, xp.astype(jnp.bfloat16), (((1,), (1,)), ((), ())),
        preferred_element_type=jnp.float32)
    e = jnp.exp2(s2 - bound2)
    eb = e.astype(jnp.bfloat16)
    den = jnp.sum(e, axis=1, keepdims=True)
    num = jax.lax.dot_general(
        eb, xb.astype(jnp.bfloat16), (((1,), (0,)), ((), ())),
        preferred_element_type=jnp.float32)
    x_node = num / den

    out = _bf16_dot(x_hyper + x_node, w_ref[...],
                    (((1,), (0,)), ((), ()))) + b_ref[...]
    o_ref[0] = out


def kernel(x, H, weight, bias, Wn_w, Wn_b, comp_w, comp_b, he_bias):
    B, N, F = x.shape
    O = weight.shape[1]
    E = H.shape[1]
    scale = 1.0 / math.sqrt(F)

    bias2 = bias.reshape(1, O)
    wnb2 = Wn_b.reshape(1, F)
    cw2 = comp_w.reshape(1, F)
    cb2 = (comp_b + he_bias).reshape(1, 1)

    grid = (B,)
    out = pl.pallas_call(
        functools.partial(_fused_body, scale=scale),
        grid=grid,
        in_specs=[
            pl.BlockSpec((1, N, F), lambda b: (b, 0, 0)),
            pl.BlockSpec((N, E), lambda b: (0, 0)),
            pl.BlockSpec((F, O), lambda b: (0, 0)),
            pl.BlockSpec((1, O), lambda b: (0, 0)),
            pl.BlockSpec((F, F), lambda b: (0, 0)),
            pl.BlockSpec((1, F), lambda b: (0, 0)),
            pl.BlockSpec((1, F), lambda b: (0, 0)),
            pl.BlockSpec((1, 1), lambda b: (0, 0)),
        ],
        out_specs=pl.BlockSpec((1, N, O), lambda b: (b, 0, 0)),
        out_shape=jax.ShapeDtypeStruct((B, N, O), jnp.float32),
        compiler_params=pltpu.CompilerParams(
            dimension_semantics=("arbitrary",),
            vmem_limit_bytes=128 * 1024 * 1024,
        ),
    )(x, H, weight, bias2, Wn_w, wnb2, cw2, cb2)
    return out
